# Initial kernel scaffold; baseline (speedup 1.0000x reference)
import jax
import jax.numpy as jnp
from jax.experimental import pallas as pl


def kernel(dec1_probs, dec2_probs, dec1_tgt, dec2_tgt, constrainer):
    raise NotImplementedError("write your pallas kernel here")


# speedup vs baseline: 1.7010x; 1.7010x over previous
import functools

import jax
import jax.numpy as jnp
from jax import lax
from jax.experimental import pallas as pl
from jax.experimental.pallas import tpu as pltpu
from jax.experimental.pallas import tpu_sc as plsc

_LANES = 16


def _sc_gather(p1_flat, p2_flat, c_flat, t1_flat, t2_flat, v1, v2, n_tok):
    info = plsc.get_sparse_core_info()
    nc, ns = info.num_cores, info.num_subcores
    nw = nc * ns
    chunk = n_tok // nw
    assert chunk % _LANES == 0 and (chunk * 4) % 8 == 0

    mesh = plsc.VectorSubcoreMesh(core_axis_name="c", subcore_axis_name="s")

    @functools.partial(
        pl.kernel,
        out_type=[
            jax.ShapeDtypeStruct((n_tok,), jnp.float32),
            jax.ShapeDtypeStruct((n_tok,), jnp.float32),
            jax.ShapeDtypeStruct((n_tok,), jnp.float32),
        ],
        mesh=mesh,
        scratch_types=[
            pltpu.VMEM((chunk,), jnp.int32),
            pltpu.VMEM((chunk,), jnp.int32),
            pltpu.VMEM((chunk,), jnp.int32),
            pltpu.VMEM((chunk,), jnp.int32),
            pltpu.VMEM((chunk,), jnp.int32),
            pltpu.VMEM((chunk,), jnp.float32),
            pltpu.VMEM((chunk,), jnp.float32),
            pltpu.VMEM((chunk,), jnp.float32),
            pltpu.SemaphoreType.DMA,
            pltpu.SemaphoreType.DMA,
            pltpu.SemaphoreType.DMA,
        ],
    )
    def k(p1_hbm, p2_hbm, c_hbm, t1_hbm, t2_hbm,
          g1_hbm, g2_hbm, gc_hbm,
          t1_v, t2_v, i1_v, i2_v, ic_v, r1_v, r2_v, rc_v,
          sem1, sem2, sem3):
        wid = lax.axis_index("s") * nc + lax.axis_index("c")
        base = wid * chunk
        pltpu.sync_copy(t1_hbm.at[pl.ds(base, chunk)], t1_v)
        pltpu.sync_copy(t2_hbm.at[pl.ds(base, chunk)], t2_v)
        for i in range(chunk // _LANES):
            sl = pl.ds(i * _LANES, _LANES)
            t1 = t1_v[sl]
            t2 = t2_v[sl]
            s1 = jnp.where(t1 == -100, 0, t1)
            s2 = jnp.where(t2 == -100, 0, t2)
            tok = base + i * _LANES + lax.broadcasted_iota(jnp.int32, (_LANES,), 0)
            i1_v[sl] = tok * v1 + s1
            i2_v[sl] = tok * v2 + s2
            ic_v[sl] = s1 * v2 + s2
        cp1 = pltpu.async_copy(p1_hbm.at[i1_v], r1_v, sem1)
        cp2 = pltpu.async_copy(p2_hbm.at[i2_v], r2_v, sem2)
        cp3 = pltpu.async_copy(c_hbm.at[ic_v], rc_v, sem3)
        cp1.wait()
        cp2.wait()
        cp3.wait()
        out_sl = pl.ds(base, chunk)
        pltpu.sync_copy(r1_v, g1_hbm.at[out_sl])
        pltpu.sync_copy(r2_v, g2_hbm.at[out_sl])
        pltpu.sync_copy(rc_v, gc_hbm.at[out_sl])

    return k(p1_flat, p2_flat, c_flat, t1_flat, t2_flat)


def _tc_loss_body(g1_ref, g2_ref, gc_ref, t1_ref, t2_ref, out_ref):
    c = jnp.clip(gc_ref[...], 0.0, 1.0)
    vv1 = g1_ref[...] * c
    vv2 = g2_ref[...] * c
    m1 = t1_ref[...] != -100
    m2 = t2_ref[...] != -100
    l1 = jnp.where(m1, -jnp.log(vv1), 0.0)
    l2 = jnp.where(m2, -jnp.log(vv2), 0.0)
    n1 = jnp.maximum(jnp.sum(m1.astype(jnp.float32)), 1.0)
    n2 = jnp.maximum(jnp.sum(m2.astype(jnp.float32)), 1.0)
    out_ref[0, 0] = jnp.sum(l1) / n1 + jnp.sum(l2) / n2


def kernel(dec1_probs, dec2_probs, dec1_tgt, dec2_tgt, constrainer):
    b, l, v1 = dec1_probs.shape
    v2 = dec2_probs.shape[2]
    n_tok = b * l

    g1, g2, gc = _sc_gather(
        dec1_probs.reshape(-1),
        dec2_probs.reshape(-1),
        constrainer.reshape(-1),
        dec1_tgt.reshape(-1),
        dec2_tgt.reshape(-1),
        v1, v2, n_tok,
    )

    rows = n_tok // 128
    out = pl.pallas_call(
        _tc_loss_body,
        out_shape=jax.ShapeDtypeStruct((1, 1), jnp.float32),
        out_specs=pl.BlockSpec(memory_space=pltpu.SMEM),
    )(
        g1.reshape(rows, 128),
        g2.reshape(rows, 128),
        gc.reshape(rows, 128),
        dec1_tgt.reshape(rows, 128),
        dec2_tgt.reshape(rows, 128),
    )
    return out[0, 0]


# speedup vs baseline: 29.5688x; 17.3836x over previous
import functools

import jax
import jax.numpy as jnp
from jax import lax
from jax.experimental import pallas as pl
from jax.experimental.pallas import tpu as pltpu
from jax.experimental.pallas import tpu_sc as plsc

_LANES = 16


def _sc_gather(p1_flat, p2_flat, c_flat, t1_flat, t2_flat, v1, v2, n_tok):
    info = plsc.get_sparse_core_info()
    nc, ns = info.num_cores, info.num_subcores
    nw = nc * ns
    chunk = n_tok // nw
    assert chunk % _LANES == 0 and (chunk * 4) % 8 == 0

    mesh = plsc.VectorSubcoreMesh(core_axis_name="c", subcore_axis_name="s")

    @functools.partial(
        pl.kernel,
        out_type=[
            jax.ShapeDtypeStruct((n_tok,), jnp.float32),
            jax.ShapeDtypeStruct((n_tok,), jnp.float32),
            jax.ShapeDtypeStruct((n_tok,), jnp.float32),
        ],
        mesh=mesh,
        scratch_types=[
            pltpu.VMEM((chunk,), jnp.int32),
            pltpu.VMEM((chunk,), jnp.int32),
            pltpu.VMEM((chunk,), jnp.int32),
            pltpu.VMEM((chunk,), jnp.int32),
            pltpu.VMEM((chunk,), jnp.int32),
            pltpu.VMEM((chunk,), jnp.float32),
            pltpu.VMEM((chunk,), jnp.float32),
            pltpu.VMEM((chunk,), jnp.float32),
            pltpu.SemaphoreType.DMA,
            pltpu.SemaphoreType.DMA,
            pltpu.SemaphoreType.DMA,
        ],
    )
    def k(p1_hbm, p2_hbm, c_hbm, t1_hbm, t2_hbm,
          g1_hbm, g2_hbm, gc_hbm,
          t1_v, t2_v, i1_v, i2_v, ic_v, r1_v, r2_v, rc_v,
          sem1, sem2, sem3):
        wid = lax.axis_index("s") * nc + lax.axis_index("c")
        base = wid * chunk
        pltpu.sync_copy(t1_hbm.at[pl.ds(base, chunk)], t1_v)
        pltpu.sync_copy(t2_hbm.at[pl.ds(base, chunk)], t2_v)
        for i in range(chunk // _LANES):
            sl = pl.ds(i * _LANES, _LANES)
            t1 = t1_v[sl]
            t2 = t2_v[sl]
            s1 = jnp.where(t1 == -100, 0, t1)
            s2 = jnp.where(t2 == -100, 0, t2)
            tok = base + i * _LANES + lax.broadcasted_iota(jnp.int32, (_LANES,), 0)
            tok_hi = lax.shift_right_logical(tok, 3)
            tok_lo = jnp.bitwise_and(tok, 7)
            i1_v[sl] = (tok_hi * (v1 * 8)
                        + lax.shift_right_logical(s1, 7) * 1024
                        + tok_lo * 128 + jnp.bitwise_and(s1, 127))
            i2_v[sl] = (tok_hi * (v2 * 8)
                        + lax.shift_right_logical(s2, 7) * 1024
                        + tok_lo * 128 + jnp.bitwise_and(s2, 127))
            ic_v[sl] = (lax.shift_right_logical(s1, 3) * (v2 * 8)
                        + lax.shift_right_logical(s2, 7) * 1024
                        + jnp.bitwise_and(s1, 7) * 128
                        + jnp.bitwise_and(s2, 127))
        cp1 = pltpu.async_copy(p1_hbm.at[i1_v], r1_v, sem1)
        cp2 = pltpu.async_copy(p2_hbm.at[i2_v], r2_v, sem2)
        cp3 = pltpu.async_copy(c_hbm.at[ic_v], rc_v, sem3)
        cp1.wait()
        cp2.wait()
        cp3.wait()
        out_sl = pl.ds(base, chunk)
        pltpu.sync_copy(r1_v, g1_hbm.at[out_sl])
        pltpu.sync_copy(r2_v, g2_hbm.at[out_sl])
        pltpu.sync_copy(rc_v, gc_hbm.at[out_sl])

    return k(p1_flat, p2_flat, c_flat, t1_flat, t2_flat)


def _tc_loss_body(g1_ref, g2_ref, gc_ref, t1_ref, t2_ref, out_ref):
    c = jnp.clip(gc_ref[...], 0.0, 1.0)
    vv1 = g1_ref[...] * c
    vv2 = g2_ref[...] * c
    m1 = t1_ref[...] != -100
    m2 = t2_ref[...] != -100
    l1 = jnp.where(m1, -jnp.log(vv1), 0.0)
    l2 = jnp.where(m2, -jnp.log(vv2), 0.0)
    n1 = jnp.maximum(jnp.sum(m1.astype(jnp.float32)), 1.0)
    n2 = jnp.maximum(jnp.sum(m2.astype(jnp.float32)), 1.0)
    out_ref[0, 0] = jnp.sum(l1) / n1 + jnp.sum(l2) / n2


def _tile_order_view(x):
    r, c = x.shape
    return x.reshape(r // 8, 8, c // 128, 128).transpose(0, 2, 1, 3).reshape(-1)


def kernel(dec1_probs, dec2_probs, dec1_tgt, dec2_tgt, constrainer):
    b, l, v1 = dec1_probs.shape
    v2 = dec2_probs.shape[2]
    n_tok = b * l

    g1, g2, gc = _sc_gather(
        _tile_order_view(dec1_probs.reshape(n_tok, v1)),
        _tile_order_view(dec2_probs.reshape(n_tok, v2)),
        _tile_order_view(constrainer),
        dec1_tgt.reshape(-1),
        dec2_tgt.reshape(-1),
        v1, v2, n_tok,
    )

    rows = n_tok // 128
    out = pl.pallas_call(
        _tc_loss_body,
        out_shape=jax.ShapeDtypeStruct((1, 1), jnp.float32),
        out_specs=pl.BlockSpec(memory_space=pltpu.SMEM),
    )(
        g1.reshape(rows, 128),
        g2.reshape(rows, 128),
        gc.reshape(rows, 128),
        dec1_tgt.reshape(rows, 128),
        dec2_tgt.reshape(rows, 128),
    )
    return out[0, 0]


# speedup vs baseline: 30.2463x; 1.0229x over previous
import functools

import jax
import jax.numpy as jnp
from jax import lax
from jax.experimental import pallas as pl
from jax.experimental.pallas import tpu as pltpu
from jax.experimental.pallas import tpu_sc as plsc

_LANES = 16


def _sc_gather(p1_flat, p2_flat, c_flat, t1_flat, t2_flat, v1, v2, n_tok):
    info = plsc.get_sparse_core_info()
    nc, ns = info.num_cores, info.num_subcores
    nw = nc * ns
    chunk = n_tok // nw
    assert chunk % _LANES == 0 and (chunk * 4) % 8 == 0

    mesh = plsc.VectorSubcoreMesh(core_axis_name="c", subcore_axis_name="s")

    @functools.partial(
        pl.kernel,
        out_type=[
            jax.ShapeDtypeStruct((n_tok,), jnp.float32),
            jax.ShapeDtypeStruct((n_tok,), jnp.float32),
            jax.ShapeDtypeStruct((n_tok,), jnp.float32),
        ],
        mesh=mesh,
        scratch_types=[
            pltpu.VMEM((chunk,), jnp.int32),
            pltpu.VMEM((chunk,), jnp.int32),
            pltpu.VMEM((chunk,), jnp.int32),
            pltpu.VMEM((chunk,), jnp.int32),
            pltpu.VMEM((chunk,), jnp.int32),
            pltpu.VMEM((chunk,), jnp.float32),
            pltpu.VMEM((chunk,), jnp.float32),
            pltpu.VMEM((chunk,), jnp.float32),
            pltpu.SemaphoreType.DMA,
            pltpu.SemaphoreType.DMA,
            pltpu.SemaphoreType.DMA,
            pltpu.SemaphoreType.DMA,
            pltpu.SemaphoreType.DMA,
        ],
    )
    def k(p1_hbm, p2_hbm, c_hbm, t1_hbm, t2_hbm,
          g1_hbm, g2_hbm, gc_hbm,
          t1_v, t2_v, i1_v, i2_v, ic_v, r1_v, r2_v, rc_v,
          sem1, sem2, sem3, sem4, sem5):
        wid = lax.axis_index("s") * nc + lax.axis_index("c")
        base = wid * chunk
        in_sl = pl.ds(base, chunk)
        ld1 = pltpu.async_copy(t1_hbm.at[in_sl], t1_v, sem4)
        ld2 = pltpu.async_copy(t2_hbm.at[in_sl], t2_v, sem5)
        iota = lax.broadcasted_iota(jnp.int32, (_LANES,), 0)
        ld1.wait()
        for i in range(chunk // _LANES):
            sl = pl.ds(i * _LANES, _LANES)
            t1 = t1_v[sl]
            s1 = jnp.where(t1 == -100, 0, t1)
            tok = base + i * _LANES + iota
            i1_v[sl] = (lax.shift_right_logical(tok, 3) * (v1 * 8)
                        + lax.shift_right_logical(s1, 7) * 1024
                        + jnp.bitwise_and(tok, 7) * 128
                        + jnp.bitwise_and(s1, 127))
        cp1 = pltpu.async_copy(p1_hbm.at[i1_v], r1_v, sem1)
        ld2.wait()
        for i in range(chunk // _LANES):
            sl = pl.ds(i * _LANES, _LANES)
            t2 = t2_v[sl]
            s2 = jnp.where(t2 == -100, 0, t2)
            tok = base + i * _LANES + iota
            i2_v[sl] = (lax.shift_right_logical(tok, 3) * (v2 * 8)
                        + lax.shift_right_logical(s2, 7) * 1024
                        + jnp.bitwise_and(tok, 7) * 128
                        + jnp.bitwise_and(s2, 127))
        cp2 = pltpu.async_copy(p2_hbm.at[i2_v], r2_v, sem2)
        for i in range(chunk // _LANES):
            sl = pl.ds(i * _LANES, _LANES)
            t1 = t1_v[sl]
            t2 = t2_v[sl]
            s1 = jnp.where(t1 == -100, 0, t1)
            s2 = jnp.where(t2 == -100, 0, t2)
            ic_v[sl] = (lax.shift_right_logical(s1, 3) * (v2 * 8)
                        + lax.shift_right_logical(s2, 7) * 1024
                        + jnp.bitwise_and(s1, 7) * 128
                        + jnp.bitwise_and(s2, 127))
        cp3 = pltpu.async_copy(c_hbm.at[ic_v], rc_v, sem3)
        out_sl = pl.ds(base, chunk)
        cp1.wait()
        st1 = pltpu.async_copy(r1_v, g1_hbm.at[out_sl], sem4)
        cp2.wait()
        st2 = pltpu.async_copy(r2_v, g2_hbm.at[out_sl], sem5)
        cp3.wait()
        pltpu.sync_copy(rc_v, gc_hbm.at[out_sl])
        st1.wait()
        st2.wait()

    return k(p1_flat, p2_flat, c_flat, t1_flat, t2_flat)


def _tc_loss_body(g1_ref, g2_ref, gc_ref, t1_ref, t2_ref, out_ref):
    c = jnp.clip(gc_ref[...], 0.0, 1.0)
    vv1 = g1_ref[...] * c
    vv2 = g2_ref[...] * c
    m1 = t1_ref[...] != -100
    m2 = t2_ref[...] != -100
    l1 = jnp.where(m1, -jnp.log(vv1), 0.0)
    l2 = jnp.where(m2, -jnp.log(vv2), 0.0)
    n1 = jnp.maximum(jnp.sum(m1.astype(jnp.float32)), 1.0)
    n2 = jnp.maximum(jnp.sum(m2.astype(jnp.float32)), 1.0)
    out_ref[0, 0] = jnp.sum(l1) / n1 + jnp.sum(l2) / n2


def _tile_order_view(x):
    r, c = x.shape
    return x.reshape(r // 8, 8, c // 128, 128).transpose(0, 2, 1, 3).reshape(-1)


def kernel(dec1_probs, dec2_probs, dec1_tgt, dec2_tgt, constrainer):
    b, l, v1 = dec1_probs.shape
    v2 = dec2_probs.shape[2]
    n_tok = b * l

    g1, g2, gc = _sc_gather(
        _tile_order_view(dec1_probs.reshape(n_tok, v1)),
        _tile_order_view(dec2_probs.reshape(n_tok, v2)),
        _tile_order_view(constrainer),
        dec1_tgt.reshape(-1),
        dec2_tgt.reshape(-1),
        v1, v2, n_tok,
    )

    rows = n_tok // 128
    out = pl.pallas_call(
        _tc_loss_body,
        out_shape=jax.ShapeDtypeStruct((1, 1), jnp.float32),
        out_specs=pl.BlockSpec(memory_space=pltpu.SMEM),
    )(
        g1.reshape(rows, 128),
        g2.reshape(rows, 128),
        gc.reshape(rows, 128),
        dec1_tgt.reshape(rows, 128),
        dec2_tgt.reshape(rows, 128),
    )
    return out[0, 0]
